# baseline (speedup 1.0000x reference)
import functools

import jax
import jax.numpy as jnp
import numpy as np
from jax import lax
from jax.experimental import pallas as pl
from jax.experimental.pallas import tpu as pltpu
from jax.experimental.pallas import tpu_sc as plsc

_H = 256
_W = 256
_RADIUS = 1.5
_WAVELENGTH = 0.1

_NC = 2
_NS = 16
_NW = _NC * _NS
_BI = 128
_NB = 128
_CHUNK = _NB * _BI
_NPAD = _NW * _CHUNK
_SBB = 32
_NSB = _NB // _SBB
_SB = _SBB * _BI
_SEG = (_H * _W) // _NS


def _tc_body(rx_ref, tx_ref, pk, att, rad, re_o, im_o, idx_o):
    def plane(k):
        return pk[0, :, pl.ds(k * _BI, _BI)]

    mx, my, mz = plane(0), plane(1), plane(2)
    c0, c1, c2, c3, c4, c5 = (plane(3), plane(4), plane(5), plane(6),
                              plane(7), plane(8))
    sr, si = plane(9), plane(10)
    dx = mx - rx_ref[0]
    dy = my - rx_ref[1]
    dz = mz - rx_ref[2]
    d_rx = jnp.sqrt(dx * dx + dy * dy + dz * dz)
    keep = (d_rx > _RADIUS).astype(jnp.float32)

    ex = mx - tx_ref[0]
    ey = my - tx_ref[1]
    ez = mz - tx_ref[2]
    d_tx = jnp.sqrt(ex * ex + ey * ey + ez * ez)
    total = d_rx + d_tx

    amp = jnp.exp(-att[0] * total) / jnp.maximum(total, 1e-6)
    phase = 2.0 * np.pi * total / _WAVELENGTH
    c = jnp.cos(phase)
    s = jnp.sin(phase)

    ssq = (c0 * c0 + c1 * c1 + c2 * c2 + c3 * c3 + c4 * c4 + c5 * c5)
    w = jnp.exp(-0.5 * ssq / (rad[0] * rad[0] + 1e-6))
    akw = amp * keep * w

    re_o[0] = akw * (sr * c - si * s)
    im_o[0] = akw * (sr * s + si * c)

    az = jnp.arctan2(dy, dx)
    zr = jnp.clip(dz / jnp.maximum(d_rx, 1e-6), -1.0, 1.0)
    el = jnp.arctan2(zr, jnp.sqrt(jnp.maximum(1.0 - zr * zr, 0.0)))
    u = jnp.clip(((az + np.pi) / (2.0 * np.pi) * _W).astype(jnp.int32),
                 0, _W - 1)
    v = jnp.clip(((el + np.pi / 2.0) / np.pi * _H).astype(jnp.int32),
                 0, _H - 1)
    idx_o[0] = v * _W + u


def _tc_stage(rx, tx, planes, att, rad):
    pspec = pl.BlockSpec((1, _NB, 11 * _BI), lambda i: (i, 0, 0))
    ispec = pl.BlockSpec((1, _NB, _BI), lambda i: (i, 0, 0))
    sspec = pl.BlockSpec(memory_space=pltpu.SMEM)
    return pl.pallas_call(
        _tc_body,
        grid=(_NW,),
        in_specs=[sspec, sspec, pspec, ispec, ispec],
        out_specs=[ispec, ispec, ispec],
        out_shape=[
            jax.ShapeDtypeStruct((_NW, _NB, _BI), jnp.float32),
            jax.ShapeDtypeStruct((_NW, _NB, _BI), jnp.float32),
            jax.ShapeDtypeStruct((_NW, _NB, _BI), jnp.int32),
        ],
        compiler_params=pltpu.CompilerParams(
            dimension_semantics=("arbitrary",)),
    )(rx, tx, planes, att, rad)


_PSB = 4096
_PNSB = _CHUNK // _PSB
_PROWS = _PSB // _BI


def _sc_prep(means_flat, cov_flat, sig_flat):
    mesh = plsc.VectorSubcoreMesh(core_axis_name="c", subcore_axis_name="s")

    @functools.partial(
        pl.kernel,
        out_type=jax.ShapeDtypeStruct((_NW, _NB, 11 * _BI), jnp.float32),
        mesh=mesh,
        scratch_types=[
            pltpu.VMEM((3 * _PSB,), jnp.float32),
            pltpu.VMEM((6 * _PSB,), jnp.float32),
            pltpu.VMEM((2 * _PSB,), jnp.float32),
            pltpu.VMEM((_PROWS, 11 * _BI), jnp.float32),
        ],
        compiler_params=pltpu.CompilerParams(use_tc_tiling_on_sc=False,
                                             needs_layout_passes=False),
    )
    def k(m_hbm, cv_hbm, sg_hbm, out_hbm, m_v, cv_v, sg_v, pk_v):
        cid = lax.axis_index("c")
        sid = lax.axis_index("s")
        wid = cid * _NS + sid
        lanes = lax.iota(jnp.int32, 16)

        def sub_batch(sb, carry):
            g0 = wid * _CHUNK + sb * _PSB
            pltpu.sync_copy(m_hbm.at[pl.ds(3 * g0, 3 * _PSB)], m_v)
            pltpu.sync_copy(cv_hbm.at[pl.ds(6 * g0, 6 * _PSB)], cv_v)
            pltpu.sync_copy(sg_hbm.at[pl.ds(2 * g0, 2 * _PSB)], sg_v)

            def grp(t, c2):
                r = t // 8
                c = (t % 8) * 16
                l16 = r * _BI + c + lanes
                for src, stride, nk, k0 in ((m_v, 3, 3, 0), (cv_v, 6, 6, 3),
                                            (sg_v, 2, 2, 9)):
                    for j in range(nk):
                        vals = plsc.load_gather(src, [stride * l16 + j])
                        pk_v[r, pl.ds((k0 + j) * _BI + c, 16)] = vals
                return c2

            lax.fori_loop(0, _PROWS * 8, grp, 0)
            pltpu.sync_copy(pk_v,
                            out_hbm.at[wid, pl.ds(sb * _PROWS, _PROWS)])
            return carry

        lax.fori_loop(0, _PNSB, sub_batch, 0)

    return k(means_flat, cov_flat, sig_flat)


def _sc_scatter(idx3, re3, im3, zeros_img):
    mesh = plsc.VectorSubcoreMesh(core_axis_name="c", subcore_axis_name="s")

    @functools.partial(
        pl.kernel,
        out_type=jax.ShapeDtypeStruct((_NC, 2 * _H * _W), jnp.float32),
        mesh=mesh,
        scratch_types=[
            pltpu.VMEM((_NB, _BI), jnp.int32),
            pltpu.VMEM((_NB, _BI), jnp.float32),
            pltpu.VMEM((_NB, _BI), jnp.float32),
            pltpu.VMEM((_SB, 8), jnp.float32),
            pltpu.VMEM((2 * _SEG,), jnp.float32),
            pltpu.VMEM_SHARED((_H * _W, 8), jnp.float32),
        ],
        compiler_params=pltpu.CompilerParams(use_tc_tiling_on_sc=False,
                                             needs_layout_passes=False),
    )
    def k(idx_hbm, re_hbm, im_hbm, z_hbm, out_hbm, idx_v, re_v, im_v, ctr_v,
          pk_v, img_sh):
        cid = lax.axis_index("c")
        sid = lax.axis_index("s")
        wid = cid * _NS + sid
        pltpu.sync_copy(z_hbm.at[pl.ds(sid * _SEG, _SEG)],
                        img_sh.at[pl.ds(sid * _SEG, _SEG)])
        pltpu.sync_copy(idx_hbm.at[wid], idx_v)
        pltpu.sync_copy(re_hbm.at[wid], re_v)
        pltpu.sync_copy(im_hbm.at[wid], im_v)
        pltpu.sync_copy(z_hbm.at[pl.ds(0, _SB)], ctr_v)
        plsc.subcore_barrier()

        lanes = lax.iota(jnp.int32, 16)
        col0 = jnp.zeros((16,), jnp.int32)
        col1 = col0 + 1

        def super_batch(sb, carry):
            def ileave(t, c2):
                r = t // 8
                c = (t % 8) * 16
                re16 = re_v[sb * _SBB + r, pl.ds(c, 16)]
                im16 = im_v[sb * _SBB + r, pl.ds(c, 16)]
                rowi = r * _BI + c + lanes
                plsc.store_scatter(ctr_v, [rowi, col0], re16)
                plsc.store_scatter(ctr_v, [rowi, col1], im16)
                return c2

            lax.fori_loop(0, _SBB * 8, ileave, 0)

            def scat(t, c2):
                pltpu.sync_copy(ctr_v.at[pl.ds(t * _BI, _BI)],
                                img_sh.at[idx_v.at[sb * _SBB + t]], add=True)
                return c2

            lax.fori_loop(0, _SBB, scat, 0)
            return carry

        lax.fori_loop(0, _NSB, super_batch, 0)
        plsc.subcore_barrier()

        pltpu.sync_copy(img_sh.at[pl.ds(sid * _SEG, _SEG)], ctr_v)

        def pack(t, c2):
            rowi = 8 * t + lanes // 2
            coli = lanes % 2
            vals = plsc.load_gather(ctr_v, [rowi, coli])
            pk_v[pl.ds(t * 16, 16)] = vals
            return c2

        lax.fori_loop(0, _SEG // 8, pack, 0)
        pltpu.sync_copy(pk_v, out_hbm.at[cid, pl.ds(sid * 2 * _SEG, 2 * _SEG)])

    return k(idx3, re3, im3, zeros_img)


def kernel(means_3d, cov3d_precomp, signal_precomp, attenuation, gaus_radii,
           rx_pos, tx_pos, bg):
    n = means_3d.shape[0]
    pad = _NPAD - n

    def col(a):
        return jnp.pad(a, (0, pad)).reshape(_NW, _NB, _BI)

    def flat(a, k):
        return jnp.pad(a.reshape(-1), (0, k * pad))

    planes = _sc_prep(flat(means_3d, 3), flat(cov3d_precomp, 6),
                      flat(signal_precomp, 2))
    re, im, idx = _tc_stage(rx_pos, tx_pos, planes,
                            col(attenuation), col(gaus_radii))

    zeros_img = jnp.zeros((_H * _W, 8), jnp.float32)
    partial = _sc_scatter(idx, re, im, zeros_img)
    img = (partial[0] + partial[1]).reshape(_H * _W, 2)
    return img.reshape(_H, _W, 2) + bg[None, None, :]


# speedup vs baseline: 7.6383x; 7.6383x over previous
import functools

import jax
import jax.numpy as jnp
import numpy as np
from jax import lax
from jax.experimental import pallas as pl
from jax.experimental.pallas import tpu as pltpu
from jax.experimental.pallas import tpu_sc as plsc

_H = 256
_W = 256
_RADIUS = 1.5
_WAVELENGTH = 0.1

_NC = 2
_NS = 16
_NW = _NC * _NS
_BI = 128
_NB = 128
_CHUNK = _NB * _BI
_NPAD = _NW * _CHUNK
_SBB = 32
_NSB = _NB // _SBB
_SB = _SBB * _BI
_SEG = (_H * _W) // _NS
_TCB = 4


def _tc_body(rx_ref, tx_ref, mx, my, mz, c0, c1, c2, c3, c4, c5, sr, si, att,
             rad, re_o, im_o, idx_o):
    dx = mx[...] - rx_ref[0]
    dy = my[...] - rx_ref[1]
    dz = mz[...] - rx_ref[2]
    d_rx = jnp.sqrt(dx * dx + dy * dy + dz * dz)
    keep = (d_rx > _RADIUS).astype(jnp.float32)

    ex = mx[...] - tx_ref[0]
    ey = my[...] - tx_ref[1]
    ez = mz[...] - tx_ref[2]
    d_tx = jnp.sqrt(ex * ex + ey * ey + ez * ez)
    total = d_rx + d_tx

    amp = jnp.exp(-att[...] * total) / jnp.maximum(total, 1e-6)
    phase = 2.0 * np.pi * total / _WAVELENGTH
    c = jnp.cos(phase)
    s = jnp.sin(phase)

    ssq = (c0[...] * c0[...] + c1[...] * c1[...] + c2[...] * c2[...] +
           c3[...] * c3[...] + c4[...] * c4[...] + c5[...] * c5[...])
    w = jnp.exp(-0.5 * ssq / (rad[...] * rad[...] + 1e-6))
    akw = amp * keep * w

    re_o[...] = akw * (sr[...] * c - si[...] * s)
    im_o[...] = akw * (sr[...] * s + si[...] * c)

    az = jnp.arctan2(dy, dx)
    zr = jnp.clip(dz / jnp.maximum(d_rx, 1e-6), -1.0, 1.0)
    el = jnp.arctan2(zr, jnp.sqrt(jnp.maximum(1.0 - zr * zr, 0.0)))
    u = jnp.clip(((az + np.pi) / (2.0 * np.pi) * _W).astype(jnp.int32),
                 0, _W - 1)
    v = jnp.clip(((el + np.pi / 2.0) / np.pi * _H).astype(jnp.int32),
                 0, _H - 1)
    idx_o[...] = v * _W + u


def _tc_stage(rx, tx, cols):
    ispec = pl.BlockSpec((_TCB, _NB, _BI), lambda i: (i, 0, 0))
    sspec = pl.BlockSpec(memory_space=pltpu.SMEM)
    return pl.pallas_call(
        _tc_body,
        grid=(_NW // _TCB,),
        in_specs=[sspec, sspec] + [ispec] * 13,
        out_specs=[ispec, ispec, ispec],
        out_shape=[
            jax.ShapeDtypeStruct((_NW, _NB, _BI), jnp.float32),
            jax.ShapeDtypeStruct((_NW, _NB, _BI), jnp.float32),
            jax.ShapeDtypeStruct((_NW, _NB, _BI), jnp.int32),
        ],
        compiler_params=pltpu.CompilerParams(
            dimension_semantics=("parallel",)),
    )(rx, tx, *cols)


def _sc_scatter(idx3, re3, im3, zeros_img):
    mesh = plsc.VectorSubcoreMesh(core_axis_name="c", subcore_axis_name="s")

    @functools.partial(
        pl.kernel,
        out_type=jax.ShapeDtypeStruct((_NC, 2 * _H * _W), jnp.float32),
        mesh=mesh,
        scratch_types=[
            pltpu.VMEM((_NB, _BI), jnp.int32),
            pltpu.VMEM((_NB, _BI), jnp.float32),
            pltpu.VMEM((_NB, _BI), jnp.float32),
            pltpu.VMEM((_SB, 8), jnp.float32),
            pltpu.VMEM((2 * _SEG,), jnp.float32),
            pltpu.VMEM_SHARED((_H * _W, 8), jnp.float32),
        ],
        compiler_params=pltpu.CompilerParams(use_tc_tiling_on_sc=False,
                                             needs_layout_passes=False),
    )
    def k(idx_hbm, re_hbm, im_hbm, z_hbm, out_hbm, idx_v, re_v, im_v, ctr_v,
          pk_v, img_sh):
        cid = lax.axis_index("c")
        sid = lax.axis_index("s")
        wid = cid * _NS + sid
        pltpu.sync_copy(z_hbm.at[pl.ds(sid * _SEG, _SEG)],
                        img_sh.at[pl.ds(sid * _SEG, _SEG)])
        pltpu.sync_copy(idx_hbm.at[wid], idx_v)
        pltpu.sync_copy(re_hbm.at[wid], re_v)
        pltpu.sync_copy(im_hbm.at[wid], im_v)
        pltpu.sync_copy(z_hbm.at[pl.ds(0, _SB)], ctr_v)
        plsc.subcore_barrier()

        lanes = lax.iota(jnp.int32, 16)
        col0 = jnp.zeros((16,), jnp.int32)
        col1 = col0 + 1

        def super_batch(sb, carry):
            def ileave(t, c2):
                r = t // 8
                c = (t % 8) * 16
                re16 = re_v[sb * _SBB + r, pl.ds(c, 16)]
                im16 = im_v[sb * _SBB + r, pl.ds(c, 16)]
                rowi = r * _BI + c + lanes
                plsc.store_scatter(ctr_v, [rowi, col0], re16)
                plsc.store_scatter(ctr_v, [rowi, col1], im16)
                return c2

            lax.fori_loop(0, _SBB * 8, ileave, 0)

            def scat(t, c2):
                pltpu.sync_copy(ctr_v.at[pl.ds(t * _BI, _BI)],
                                img_sh.at[idx_v.at[sb * _SBB + t]], add=True)
                return c2

            lax.fori_loop(0, _SBB, scat, 0)
            return carry

        lax.fori_loop(0, _NSB, super_batch, 0)
        plsc.subcore_barrier()

        pltpu.sync_copy(img_sh.at[pl.ds(sid * _SEG, _SEG)], ctr_v)

        def pack(t, c2):
            rowi = 8 * t + lanes // 2
            coli = lanes % 2
            vals = plsc.load_gather(ctr_v, [rowi, coli])
            pk_v[pl.ds(t * 16, 16)] = vals
            return c2

        lax.fori_loop(0, _SEG // 8, pack, 0)
        pltpu.sync_copy(pk_v, out_hbm.at[cid, pl.ds(sid * 2 * _SEG, 2 * _SEG)])

    return k(idx3, re3, im3, zeros_img)


def kernel(means_3d, cov3d_precomp, signal_precomp, attenuation, gaus_radii,
           rx_pos, tx_pos, bg):
    n = means_3d.shape[0]
    pad = _NPAD - n

    def col(a):
        return jnp.pad(a, (0, pad)).reshape(_NW, _NB, _BI)

    cols = (
        [col(means_3d[:, i]) for i in range(3)]
        + [col(cov3d_precomp[:, i]) for i in range(6)]
        + [col(signal_precomp[:, i]) for i in range(2)]
        + [col(attenuation), col(gaus_radii)]
    )
    re, im, idx = _tc_stage(rx_pos, tx_pos, cols)

    zeros_img = jnp.zeros((_H * _W, 8), jnp.float32)
    partial = _sc_scatter(idx, re, im, zeros_img)
    img = (partial[0] + partial[1]).reshape(_H * _W, 2)
    return img.reshape(_H, _W, 2) + bg[None, None, :]
